# baseline (speedup 1.0000x reference)
import functools

import jax
import jax.numpy as jnp
from jax import lax
from jax.experimental import pallas as pl
from jax.experimental.pallas import tpu as pltpu
from jax.experimental.pallas import tpu_sc as plsc


def _project_body(table_ref, w_ref, b_ref, out_ref):
    out_ref[...] = (
        lax.dot_general(
            table_ref[...],
            w_ref[...],
            (((1,), (1,)), ((), ())),
            preferred_element_type=jnp.float32,
        )
        + b_ref[...]
    )


def _project(table_padded, W, b2d):
    vp, d = table_padded.shape
    return pl.pallas_call(
        _project_body,
        out_shape=jax.ShapeDtypeStruct((vp, d), jnp.float32),
    )(table_padded, W, b2d)


def _make_gather(vp, d, batch):
    info = plsc.get_sparse_core_info()
    nc, ns = info.num_cores, info.num_subcores
    nw = nc * ns
    assert batch % nw == 0
    b_per_w = batch // nw
    chunk = 64
    n_chunks = b_per_w // chunk
    assert b_per_w % chunk == 0

    mesh = plsc.VectorSubcoreMesh(core_axis_name="c", subcore_axis_name="s")

    @functools.partial(
        pl.kernel,
        mesh=mesh,
        out_type=jax.ShapeDtypeStruct((batch, d), jnp.float32),
        scratch_types=[
            pltpu.VMEM((b_per_w,), jnp.int32),
            pltpu.VMEM((2, chunk, d), jnp.float32),
            pltpu.SemaphoreType.DMA,
            pltpu.SemaphoreType.DMA,
            pltpu.SemaphoreType.DMA,
            pltpu.SemaphoreType.DMA,
        ],
    )
    def gather_kernel(proj_hbm, idx_hbm, out_hbm, idx_v, bufs, g0, g1, w0, w1):
        wid = lax.axis_index("s") * nc + lax.axis_index("c")
        base = wid * b_per_w
        pltpu.sync_copy(idx_hbm.at[pl.ds(base, b_per_w)], idx_v)
        gsem = (g0, g1)
        wsem = (w0, w1)

        def gstart(j):
            return pltpu.async_copy(
                proj_hbm.at[idx_v.at[pl.ds(j * chunk, chunk)]],
                bufs.at[j % 2],
                gsem[j % 2],
            )

        gathers = [gstart(0)]
        writes = [None, None]
        for j in range(n_chunks):
            gathers[j].wait()
            if writes[(j + 1) % 2] is not None:
                writes[(j + 1) % 2].wait()
            if j + 1 < n_chunks:
                gathers.append(gstart(j + 1))
            writes[j % 2] = pltpu.async_copy(
                bufs.at[j % 2],
                out_hbm.at[pl.ds(base + j * chunk, chunk)],
                wsem[j % 2],
            )
        writes[(n_chunks - 1) % 2].wait()

    return gather_kernel


def kernel(expert_id, table, W, b):
    v, d = table.shape
    (batch,) = expert_id.shape
    vp = (v + 7) // 8 * 8
    table_padded = jnp.pad(table, ((0, vp - v), (0, 0)))
    ids = expert_id.astype(jnp.int32)
    out = _make_gather(vp, d, batch)(table_padded, ids)
    return out


# speedup vs baseline: 1.2103x; 1.2103x over previous
import functools

import jax
import jax.numpy as jnp
from jax import lax
from jax.experimental import pallas as pl
from jax.experimental.pallas import tpu as pltpu
from jax.experimental.pallas import tpu_sc as plsc


def _project_body(table_ref, w_ref, b_ref, out_ref):
    out_ref[...] = (
        lax.dot_general(
            table_ref[...],
            w_ref[...],
            (((1,), (1,)), ((), ())),
            preferred_element_type=jnp.float32,
        )
        + b_ref[...]
    )


def _project(table_padded, W, b2d):
    vp, d = table_padded.shape
    return pl.pallas_call(
        _project_body,
        out_shape=jax.ShapeDtypeStruct((vp, d), jnp.float32),
    )(table_padded, W, b2d)


def _make_gather(v, d, batch):
    info = plsc.get_sparse_core_info()
    nc, ns = info.num_cores, info.num_subcores
    nw = nc * ns
    assert batch % nw == 0
    b_per_w = batch // nw
    rows_per_buf = 2
    nbuf = 2
    n_chunks = b_per_w // rows_per_buf
    outer = n_chunks // nbuf

    mesh = plsc.VectorSubcoreMesh(core_axis_name="c", subcore_axis_name="s")

    @functools.partial(
        pl.kernel,
        mesh=mesh,
        out_type=jax.ShapeDtypeStruct((batch, d), jnp.float32),
        scratch_types=[
            pltpu.VMEM((b_per_w,), jnp.int32),
            pltpu.VMEM((v, d), jnp.float32),
            pltpu.SemaphoreType.DMA,
            pltpu.SemaphoreType.DMA,
        ],
    )
    def gather_kernel(proj_hbm, idx_hbm, out_hbm, idx_s, tab_v, tsem, wsem):
        wid = lax.axis_index("s") * nc + lax.axis_index("c")
        base = wid * b_per_w
        tstage = pltpu.async_copy(proj_hbm, tab_v, tsem)
        pltpu.sync_copy(idx_hbm.at[pl.ds(base, b_per_w)], idx_s)
        tstage.wait()

        n_groups = b_per_w // 16

        def body(jo, carry):
            ids16 = idx_s[pl.ds(jo * 16, 16)]
            for k in range(16):
                row = ids16[k]
                pltpu.async_copy(
                    tab_v.at[row],
                    out_hbm.at[base + jo * 16 + k],
                    wsem,
                )
            return carry

        lax.fori_loop(0, n_groups, body, 0)
        for _ in range(n_groups):
            pltpu.make_async_copy(
                tab_v.at[pl.ds(0, 16)],
                out_hbm.at[pl.ds(base, 16)],
                wsem,
            ).wait()

    return gather_kernel


def kernel(expert_id, table, W, b):
    v, d = table.shape
    (batch,) = expert_id.shape
    vp = (v + 7) // 8 * 8
    table_padded = jnp.pad(table, ((0, vp - v), (0, 0)))
    projected = _project(table_padded, W, b.reshape(1, d))[:v]
    ids = expert_id.astype(jnp.int32)
    out = _make_gather(v, d, batch)(projected, ids)
    return out


# speedup vs baseline: 1.2376x; 1.0226x over previous
import functools

import jax
import jax.numpy as jnp
from jax import lax
from jax.experimental import pallas as pl
from jax.experimental.pallas import tpu as pltpu
from jax.experimental.pallas import tpu_sc as plsc


def _project_body(table_ref, w_ref, b_ref, out_ref):
    out_ref[...] = (
        lax.dot_general(
            table_ref[...],
            w_ref[...],
            (((1,), (1,)), ((), ())),
            preferred_element_type=jnp.float32,
        )
        + b_ref[...]
    )


def _project(table_padded, W, b2d):
    vp, d = table_padded.shape
    return pl.pallas_call(
        _project_body,
        out_shape=jax.ShapeDtypeStruct((vp, d), jnp.float32),
    )(table_padded, W, b2d)


def _make_gather(v, d, batch):
    info = plsc.get_sparse_core_info()
    nc, ns = 1, info.num_subcores
    nw = nc * ns
    assert batch % nw == 0
    b_per_w = batch // nw

    mesh = plsc.VectorSubcoreMesh(
        core_axis_name="c", subcore_axis_name="s", num_cores=nc
    )

    @functools.partial(
        pl.kernel,
        mesh=mesh,
        out_type=jax.ShapeDtypeStruct((batch, d), jnp.float32),
        scratch_types=[
            pltpu.VMEM((b_per_w,), jnp.int32),
            pltpu.VMEM((v, d), jnp.float32),
            pltpu.SemaphoreType.DMA,
            pltpu.SemaphoreType.DMA,
        ],
    )
    def gather_kernel(proj_hbm, idx_hbm, out_hbm, idx_s, tab_v, tsem, wsem):
        wid = lax.axis_index("s") * nc + lax.axis_index("c")
        base = wid * b_per_w
        tstage = pltpu.async_copy(proj_hbm, tab_v, tsem)
        pltpu.sync_copy(idx_hbm.at[pl.ds(base, b_per_w)], idx_s)
        tstage.wait()

        n_groups = b_per_w // 16

        def body(jo, carry):
            ids16 = idx_s[pl.ds(jo * 16, 16)]
            for k in range(16):
                row = ids16[k]
                pltpu.async_copy(
                    tab_v.at[row],
                    out_hbm.at[base + jo * 16 + k],
                    wsem,
                )
            return carry

        lax.fori_loop(0, n_groups, body, 0)
        for _ in range(n_groups):
            pltpu.make_async_copy(
                tab_v.at[pl.ds(0, 16)],
                out_hbm.at[pl.ds(base, 16)],
                wsem,
            ).wait()

    return gather_kernel


def kernel(expert_id, table, W, b):
    v, d = table.shape
    (batch,) = expert_id.shape
    vp = (v + 7) // 8 * 8
    table_padded = jnp.pad(table, ((0, vp - v), (0, 0)))
    projected = _project(table_padded, W, b.reshape(1, d))[:v]
    ids = expert_id.astype(jnp.int32)
    out = _make_gather(v, d, batch)(projected, ids)
    return out
